# baseline (speedup 1.0000x reference)
import functools

import jax
import jax.numpy as jnp
from jax import lax
from jax.experimental import pallas as pl
from jax.experimental.pallas import tpu as pltpu
from jax.experimental.pallas import tpu_sc as plsc

B = 4
T = 2048
H = 1024
NC = 2
NS = 16
NW = NC * NS
P = T // NW
CH = 32
NSL = H // 16
EPS = 1e-12


def _lane_sum(v):
    dnums = lax.GatherDimensionNumbers(
        offset_dims=(), collapsed_slice_dims=(0,), start_index_map=(0,))
    for sh in (1, 2, 4, 8):
        idx = jnp.arange(16, dtype=jnp.int32) ^ sh
        v = v + lax.gather(v, idx[:, None], dnums, slice_sizes=(1,),
                           mode=lax.GatherScatterMode.PROMISE_IN_BOUNDS)
    return v


def _token_ln(word_v, pos_v, gb_v, pos_off, i):
    acc = jnp.zeros((16,), jnp.float32)
    acc2 = jnp.zeros((16,), jnp.float32)
    for k in range(NSL):
        sl = pl.ds(k * 16, 16)
        v = word_v[i, sl] + pos_v[pos_off + i, sl]
        word_v[i, sl] = v
        acc = acc + v
        acc2 = acc2 + v * v
    mean_v = _lane_sum(acc) * (1.0 / H)
    var_v = _lane_sum(acc2) * (1.0 / H) - mean_v * mean_v
    x = var_v + EPS
    xi = lax.bitcast_convert_type(x, jnp.int32)
    yi = jnp.full((16,), 0x5F3759DF, jnp.int32) - (xi >> 1)
    y = lax.bitcast_convert_type(yi, jnp.float32)
    for _ in range(3):
        y = y * (1.5 - 0.5 * x * y * y)
    rstd_v = y
    for k in range(NSL):
        sl = pl.ds(k * 16, 16)
        v = word_v[i, sl]
        g = gb_v[0, sl]
        bb = gb_v[1, sl]
        word_v[i, sl] = (v - mean_v) * rstd_v * g + bb
    return i


@functools.partial(
    pl.kernel,
    mesh=plsc.VectorSubcoreMesh(core_axis_name="c", subcore_axis_name="s"),
    out_type=jax.ShapeDtypeStruct((B * T, H), jnp.float32),
    scratch_types=[
        pltpu.VMEM((CH,), jnp.int32),
        pltpu.VMEM((P, H), jnp.float32),
        pltpu.VMEM((CH, H), jnp.float32),
        pltpu.VMEM((2, H), jnp.float32),
        pltpu.SemaphoreType.DMA,
    ],
)
def _sc_embed_ln(ids_hbm, wemb_hbm, pemb_hbm, gamma_hbm, beta_hbm, out_hbm,
                 idx_v, pos_v, word_v, gb_v, sem):
    c = lax.axis_index("c")
    s = lax.axis_index("s")
    wid = s * NC + c
    pbase = wid * P
    pltpu.sync_copy(gamma_hbm, gb_v.at[0])
    pltpu.sync_copy(beta_hbm, gb_v.at[1])
    pltpu.sync_copy(pemb_hbm.at[pl.ds(pbase, P)], pos_v)
    for b in range(B):
        for ch in range(P // CH):
            tok0 = b * T + pbase + ch * CH
            pltpu.sync_copy(ids_hbm.at[pl.ds(tok0, CH)], idx_v)
            pltpu.async_copy(wemb_hbm.at[idx_v], word_v, sem).wait()
            body = functools.partial(_token_ln, word_v, pos_v, gb_v, ch * CH)
            lax.fori_loop(0, CH, lambda i, _: body(i), 0)
            pltpu.sync_copy(word_v, out_hbm.at[pl.ds(tok0, CH)])


def kernel(input_ids, word_emb, pos_emb, gamma, beta):
    ids_flat = input_ids.reshape(-1).astype(jnp.int32)
    out = _sc_embed_ln(ids_flat, word_emb, pos_emb, gamma, beta)
    return out.reshape(B, T, H)


# speedup vs baseline: 1.0227x; 1.0227x over previous
import functools

import jax
import jax.numpy as jnp
from jax import lax
from jax.experimental import pallas as pl
from jax.experimental.pallas import tpu as pltpu
from jax.experimental.pallas import tpu_sc as plsc

B = 4
T = 2048
H = 1024
NC = 2
NS = 16
NW = NC * NS
P = T // NW
CH = 16
NG = P // CH
NSLOT = B
NSL = H // 16
EPS = 1e-12


def _lane_sum(v):
    dnums = lax.GatherDimensionNumbers(
        offset_dims=(), collapsed_slice_dims=(0,), start_index_map=(0,))
    for sh in (1, 2, 4, 8):
        idx = jnp.arange(16, dtype=jnp.int32) ^ sh
        v = v + lax.gather(v, idx[:, None], dnums, slice_sizes=(1,),
                           mode=lax.GatherScatterMode.PROMISE_IN_BOUNDS)
    return v


def _token_ln(word_v, pos_v, gb_v, par, i):
    accs = [jnp.zeros((16,), jnp.float32) for _ in range(4)]
    acc2s = [jnp.zeros((16,), jnp.float32) for _ in range(4)]
    for k in range(NSL):
        sl = pl.ds(k * 16, 16)
        v = word_v[par, i, sl] + pos_v[i, sl]
        word_v[par, i, sl] = v
        accs[k % 4] = accs[k % 4] + v
        acc2s[k % 4] = acc2s[k % 4] + v * v
    acc = (accs[0] + accs[1]) + (accs[2] + accs[3])
    acc2 = (acc2s[0] + acc2s[1]) + (acc2s[2] + acc2s[3])
    mean_v = _lane_sum(acc) * (1.0 / H)
    var_v = _lane_sum(acc2) * (1.0 / H) - mean_v * mean_v
    x = var_v + EPS
    xi = lax.bitcast_convert_type(x, jnp.int32)
    yi = jnp.full((16,), 0x5F3759DF, jnp.int32) - (xi >> 1)
    y = lax.bitcast_convert_type(yi, jnp.float32)
    for _ in range(3):
        y = y * (1.5 - 0.5 * x * y * y)
    rstd_v = y
    for k in range(NSL):
        sl = pl.ds(k * 16, 16)
        v = word_v[par, i, sl]
        g = gb_v[0, sl]
        bb = gb_v[1, sl]
        word_v[par, i, sl] = (v - mean_v) * rstd_v * g + bb
    return i


@functools.partial(
    pl.kernel,
    mesh=plsc.VectorSubcoreMesh(core_axis_name="c", subcore_axis_name="s"),
    out_type=jax.ShapeDtypeStruct((B * T, H), jnp.float32),
    scratch_types=[
        pltpu.VMEM((B * P,), jnp.int32),
        pltpu.VMEM((CH, H), jnp.float32),
        pltpu.VMEM((NSLOT, CH, H), jnp.float32),
        pltpu.VMEM((2, H), jnp.float32),
        pltpu.SemaphoreType.DMA,
        pltpu.SemaphoreType.DMA,
        pltpu.SemaphoreType.DMA,
        pltpu.SemaphoreType.DMA,
        pltpu.SemaphoreType.DMA,
        pltpu.SemaphoreType.DMA,
        pltpu.SemaphoreType.DMA,
        pltpu.SemaphoreType.DMA,
    ],
)
def _sc_embed_ln(ids_hbm, wemb_hbm, pemb_hbm, gamma_hbm, beta_hbm, out_hbm,
                 idx_v, pos_v, word_v, gb_v,
                 sg0, sg1, sg2, sg3, so0, so1, so2, so3):
    sg = (sg0, sg1, sg2, sg3)
    so = (so0, so1, so2, so3)
    c = lax.axis_index("c")
    s = lax.axis_index("s")
    wid = s * NC + c
    pbase = wid * P

    pltpu.sync_copy(gamma_hbm, gb_v.at[0])
    pltpu.sync_copy(beta_hbm, gb_v.at[1])
    for b in range(B):
        pltpu.sync_copy(ids_hbm.at[pl.ds(b * T + pbase, P)],
                        idx_v.at[pl.ds(b * P, P)])

    def gather_issue(g, par):
        ioff = par * P + g * CH
        pltpu.async_copy(wemb_hbm.at[idx_v.at[pl.ds(ioff, CH)]],
                         word_v.at[par], sg[par])

    def gather_wait(g, par):
        ioff = par * P + g * CH
        pltpu.make_async_copy(wemb_hbm.at[idx_v.at[pl.ds(ioff, CH)]],
                              word_v.at[par], sg[par]).wait()

    def out_issue(g, par):
        tok0 = par * T + pbase + g * CH
        pltpu.async_copy(word_v.at[par], out_hbm.at[pl.ds(tok0, CH)], so[par])

    def out_wait(par):
        pltpu.make_async_copy(word_v.at[par], out_hbm.at[pl.ds(0, CH)],
                              so[par]).wait()

    for par in range(NSLOT):
        gather_issue(0, par)

    def group(g, carry):
        pltpu.sync_copy(pemb_hbm.at[pl.ds(pbase + g * CH, CH)], pos_v)
        for par in range(NSLOT):
            gather_wait(g, par)
            body = functools.partial(_token_ln, word_v, pos_v, gb_v, par)
            lax.fori_loop(0, CH, lambda i, _: body(i), 0)
            out_issue(g, par)
            if par > 0:
                out_wait(par - 1)

                @pl.when(g < NG - 1)
                def _():
                    gather_issue(g + 1, par - 1)
        out_wait(NSLOT - 1)

        @pl.when(g < NG - 1)
        def _():
            gather_issue(g + 1, NSLOT - 1)

        return carry

    lax.fori_loop(0, NG, group, 0)


def kernel(input_ids, word_emb, pos_emb, gamma, beta):
    ids_flat = input_ids.reshape(-1).astype(jnp.int32)
    out = _sc_embed_ln(ids_flat, word_emb, pos_emb, gamma, beta)
    return out.reshape(B, T, H)


# speedup vs baseline: 2.3596x; 2.3073x over previous
import functools

import jax
import jax.numpy as jnp
from jax import lax
from jax.experimental import pallas as pl
from jax.experimental.pallas import tpu as pltpu
from jax.experimental.pallas import tpu_sc as plsc

B = 4
T = 2048
H = 1024
NC = 2
NS = 16
NW = NC * NS
TOK = B * T
PW = TOK // NW
CH = 32
NCHUNK = PW // CH
NSLOT = 3
BT = 256
EPS = 1e-12


@functools.partial(
    pl.kernel,
    mesh=plsc.VectorSubcoreMesh(core_axis_name="c", subcore_axis_name="s"),
    out_type=jax.ShapeDtypeStruct((TOK, H), jnp.float32),
    scratch_types=[
        pltpu.VMEM((PW,), jnp.int32),
        pltpu.VMEM((NSLOT, CH, H), jnp.float32),
        pltpu.SemaphoreType.DMA,
        pltpu.SemaphoreType.DMA,
        pltpu.SemaphoreType.DMA,
        pltpu.SemaphoreType.DMA,
        pltpu.SemaphoreType.DMA,
        pltpu.SemaphoreType.DMA,
    ],
)
def _sc_gather(ids_hbm, wemb_hbm, out_hbm, idx_v, rows_v,
               sg0, sg1, sg2, so0, so1, so2):
    sg = (sg0, sg1, sg2)
    so = (so0, so1, so2)
    c = lax.axis_index("c")
    s = lax.axis_index("s")
    wid = s * NC + c
    base = wid * PW

    pltpu.sync_copy(ids_hbm.at[pl.ds(base, PW)], idx_v)

    def gather_issue(j):
        pltpu.async_copy(wemb_hbm.at[idx_v.at[pl.ds(j * CH, CH)]],
                         rows_v.at[j % NSLOT], sg[j % NSLOT])

    def gather_wait(j):
        pltpu.make_async_copy(wemb_hbm.at[idx_v.at[pl.ds(j * CH, CH)]],
                              rows_v.at[j % NSLOT], sg[j % NSLOT]).wait()

    def out_issue(j):
        pltpu.async_copy(rows_v.at[j % NSLOT],
                         out_hbm.at[pl.ds(base + j * CH, CH)], so[j % NSLOT])

    def out_wait(j):
        pltpu.make_async_copy(rows_v.at[j % NSLOT],
                              out_hbm.at[pl.ds(base + j * CH, CH)],
                              so[j % NSLOT]).wait()

    gather_issue(0)
    gather_issue(1)
    for j in range(NCHUNK):
        if j + 2 < NCHUNK:
            if j >= 1:
                out_wait(j - 1)
            gather_issue(j + 2)
        gather_wait(j)
        out_issue(j)
    out_wait(NCHUNK - 2)
    out_wait(NCHUNK - 1)


def _tc_ln(emb_ref, pos_ref, g_ref, b_ref, o_ref):
    x = emb_ref[0] + pos_ref[...]
    mean = jnp.mean(x, axis=-1, keepdims=True)
    xc = x - mean
    var = jnp.mean(xc * xc, axis=-1, keepdims=True)
    o_ref[0] = (xc * lax.rsqrt(var + EPS)) * g_ref[...] + b_ref[...]


def kernel(input_ids, word_emb, pos_emb, gamma, beta):
    ids_flat = input_ids.reshape(-1).astype(jnp.int32)
    gathered = _sc_gather(ids_flat, word_emb).reshape(B, T, H)
    out = pl.pallas_call(
        _tc_ln,
        grid=(B, T // BT),
        in_specs=[
            pl.BlockSpec((1, BT, H), lambda b, j: (b, j, 0)),
            pl.BlockSpec((BT, H), lambda b, j: (j, 0)),
            pl.BlockSpec((1, H), lambda b, j: (0, 0)),
            pl.BlockSpec((1, H), lambda b, j: (0, 0)),
        ],
        out_specs=pl.BlockSpec((1, BT, H), lambda b, j: (b, j, 0)),
        out_shape=jax.ShapeDtypeStruct((B, T, H), jnp.float32),
    )(gathered, pos_emb, gamma.reshape(1, H), beta.reshape(1, H))
    return out


# speedup vs baseline: 2.6992x; 1.1439x over previous
import functools

import jax
import jax.numpy as jnp
from jax import lax
from jax.experimental import pallas as pl
from jax.experimental.pallas import tpu as pltpu
from jax.experimental.pallas import tpu_sc as plsc

B = 4
T = 2048
H = 1024
NC = 2
NS = 16
NW = NC * NS
TOK = B * T
PW = TOK // NW
CH = 32
NCHUNK = PW // CH
NSLOT = 3
BT = 512
EPS = 1e-12


@functools.partial(
    pl.kernel,
    mesh=plsc.VectorSubcoreMesh(core_axis_name="c", subcore_axis_name="s"),
    out_type=jax.ShapeDtypeStruct((TOK, H), jnp.float32),
    scratch_types=[
        pltpu.VMEM((PW,), jnp.int32),
        pltpu.VMEM((NSLOT, CH, H), jnp.float32),
        pltpu.SemaphoreType.DMA,
        pltpu.SemaphoreType.DMA,
        pltpu.SemaphoreType.DMA,
        pltpu.SemaphoreType.DMA,
        pltpu.SemaphoreType.DMA,
        pltpu.SemaphoreType.DMA,
    ],
)
def _sc_gather(ids_hbm, wemb_hbm, out_hbm, idx_v, rows_v,
               sg0, sg1, sg2, so0, so1, so2):
    sg = (sg0, sg1, sg2)
    so = (so0, so1, so2)
    c = lax.axis_index("c")
    s = lax.axis_index("s")
    wid = s * NC + c
    base = wid * PW

    pltpu.sync_copy(ids_hbm.at[pl.ds(base, PW)], idx_v)

    def gather_issue(j):
        pltpu.async_copy(wemb_hbm.at[idx_v.at[pl.ds(j * CH, CH)]],
                         rows_v.at[j % NSLOT], sg[j % NSLOT])

    def gather_wait(j):
        pltpu.make_async_copy(wemb_hbm.at[idx_v.at[pl.ds(j * CH, CH)]],
                              rows_v.at[j % NSLOT], sg[j % NSLOT]).wait()

    def out_issue(j):
        pltpu.async_copy(rows_v.at[j % NSLOT],
                         out_hbm.at[pl.ds(base + j * CH, CH)], so[j % NSLOT])

    def out_wait(j):
        pltpu.make_async_copy(rows_v.at[j % NSLOT],
                              out_hbm.at[pl.ds(base + j * CH, CH)],
                              so[j % NSLOT]).wait()

    gather_issue(0)
    gather_issue(1)
    for j in range(NCHUNK):
        if j + 2 < NCHUNK:
            if j >= 1:
                out_wait(j - 1)
            gather_issue(j + 2)
        gather_wait(j)
        out_issue(j)
    out_wait(NCHUNK - 2)
    out_wait(NCHUNK - 1)


def _tc_ln(emb_ref, pos_ref, g_ref, b_ref, o_ref):
    x = emb_ref[0] + pos_ref[...]
    mean = jnp.mean(x, axis=-1, keepdims=True)
    xc = x - mean
    var = jnp.mean(xc * xc, axis=-1, keepdims=True)
    o_ref[0] = (xc * lax.rsqrt(var + EPS)) * g_ref[...] + b_ref[...]


def kernel(input_ids, word_emb, pos_emb, gamma, beta):
    ids_flat = input_ids.reshape(-1).astype(jnp.int32)
    gathered = _sc_gather(ids_flat, word_emb).reshape(B, T, H)
    out = pl.pallas_call(
        _tc_ln,
        grid=(T // BT, B),
        in_specs=[
            pl.BlockSpec((1, BT, H), lambda j, b: (b, j, 0)),
            pl.BlockSpec((BT, H), lambda j, b: (j, 0)),
            pl.BlockSpec((1, H), lambda j, b: (0, 0)),
            pl.BlockSpec((1, H), lambda j, b: (0, 0)),
        ],
        out_specs=pl.BlockSpec((1, BT, H), lambda j, b: (b, j, 0)),
        out_shape=jax.ShapeDtypeStruct((B, T, H), jnp.float32),
    )(gathered, pos_emb, gamma.reshape(1, H), beta.reshape(1, H))
    return out
